# Initial kernel scaffold; baseline (speedup 1.0000x reference)
import jax
import jax.numpy as jnp
from jax.experimental import pallas as pl


def kernel(x_i, x_j, inner_edge_index_i, inner_edge_index_j, outer_edge_index_i, outer_edge_index_j, x_i_batch, x_j_batch, W_enc, b_enc, W_inner, b_inner, W_outer, b_outer, W_update, b_update, W_read, b_read):
    raise NotImplementedError("write your pallas kernel here")


# speedup vs baseline: 2.0694x; 2.0694x over previous
import jax
import jax.numpy as jnp
from jax import lax
from jax.experimental import pallas as pl
from jax.experimental.pallas import tpu as pltpu
from jax.experimental.pallas import tpu_sc as plsc

_N = 10000
_D = 128
_E = 320000
_B = 128

_NP = 10240
_SC_CORES = 2
_SC_TILES = 16
_NW = _SC_CORES * _SC_TILES
_K = 128
_EP = 327680
_EPT = _EP // _NW
_NCH = _EPT // _K
_SEG = 8
_RPT = _NP // _SC_TILES

_mesh = plsc.VectorSubcoreMesh(core_axis_name="c", subcore_axis_name="s")


def _mp_body(x_hbm, src_hbm, dst_hbm, out_hbm,
             src_v, dst_v, rows_a, rows_b, acc_sh, sem_a, sem_b):
    cid = lax.axis_index("c")
    sid = lax.axis_index("s")
    wid = cid * _SC_TILES + sid

    def _zb(i, carry):
        rows_a[i // 8, pl.ds((i % 8) * 16, 16)] = jnp.zeros((16,), jnp.float32)
        return carry
    lax.fori_loop(0, _K * 8, _zb, 0)
    for t in range(_RPT // _K):
        pltpu.sync_copy(rows_a, acc_sh.at[pl.ds(sid * _RPT + t * _K, _K)])

    plsc.subcore_barrier()

    for s in range((_NCH + _SEG - 1) // _SEG):
        n = min(_SEG, _NCH - s * _SEG)
        pltpu.sync_copy(src_hbm.at[wid, pl.ds(s * _SEG, n)], src_v.at[pl.ds(0, n)])
        pltpu.sync_copy(dst_hbm.at[wid, pl.ds(s * _SEG, n)], dst_v.at[pl.ds(0, n)])

        pltpu.async_copy(x_hbm.at[src_v.at[0]], rows_a, sem_a)

        def _chunk(i, carry):
            ja = 2 * i
            jb = ja + 1
            pltpu.async_copy(x_hbm.at[src_v.at[jb]], rows_b, sem_b)
            pltpu.make_async_copy(x_hbm.at[src_v.at[ja]], rows_a, sem_a).wait()
            pltpu.sync_copy(rows_a, acc_sh.at[dst_v.at[ja]], add=True)

            @pl.when(jb + 1 < n)
            def _():
                pltpu.async_copy(x_hbm.at[src_v.at[jb + 1]], rows_a, sem_a)

            pltpu.make_async_copy(x_hbm.at[src_v.at[jb]], rows_b, sem_b).wait()
            pltpu.sync_copy(rows_b, acc_sh.at[dst_v.at[jb]], add=True)
            return carry

        lax.fori_loop(0, (n - 1) // 2, _chunk, 0)
        if n % 2 == 1:
            pltpu.make_async_copy(x_hbm.at[src_v.at[n - 1]], rows_a, sem_a).wait()
            pltpu.sync_copy(rows_a, acc_sh.at[dst_v.at[n - 1]], add=True)
        else:
            pltpu.async_copy(x_hbm.at[src_v.at[n - 1]], rows_b, sem_b)
            pltpu.make_async_copy(x_hbm.at[src_v.at[n - 2]], rows_a, sem_a).wait()
            pltpu.sync_copy(rows_a, acc_sh.at[dst_v.at[n - 2]], add=True)
            pltpu.make_async_copy(x_hbm.at[src_v.at[n - 1]], rows_b, sem_b).wait()
            pltpu.sync_copy(rows_b, acc_sh.at[dst_v.at[n - 1]], add=True)

    plsc.subcore_barrier()

    base = sid * _RPT
    pltpu.sync_copy(acc_sh.at[pl.ds(base, _RPT)],
                    out_hbm.at[cid, pl.ds(base, _RPT)])


_mp_call = pl.kernel(
    _mp_body,
    out_type=jax.ShapeDtypeStruct((_SC_CORES, _NP, _D), jnp.float32),
    mesh=_mesh,
    scratch_types=[
        pltpu.VMEM((_SEG, _K), jnp.int32),
        pltpu.VMEM((_SEG, _K), jnp.int32),
        pltpu.VMEM((_K, _D), jnp.float32),
        pltpu.VMEM((_K, _D), jnp.float32),
        pltpu.VMEM_SHARED((_NP, _D), jnp.float32),
        pltpu.SemaphoreType.DMA,
        pltpu.SemaphoreType.DMA,
    ],
)


def _edges(edge_index):
    src = jnp.pad(edge_index[0].astype(jnp.int32), (0, _EP - _E))
    dst = jnp.pad(edge_index[1].astype(jnp.int32), (0, _EP - _E),
                  constant_values=_NP - 1)
    return src.reshape(_NW, _NCH, _K), dst.reshape(_NW, _NCH, _K)


def _mp(x, edges):
    return _mp_call(x, edges[0], edges[1])


def _lrelu(x):
    return jnp.where(x >= 0, x, 0.01 * x)


_BLK = 2048


def _enc_body(a_ref, w_ref, b_ref, o_ref):
    a = a_ref[0] + a_ref[1]
    o_ref[...] = jnp.dot(a, w_ref[...], preferred_element_type=jnp.float32) + b_ref[...]


@jax.jit
def _enc(agg, w, b):
    return pl.pallas_call(
        _enc_body,
        grid=(_NP // _BLK,),
        in_specs=[
            pl.BlockSpec((_SC_CORES, _BLK, _D), lambda i: (0, i, 0)),
            pl.BlockSpec((_D, _D), lambda i: (0, 0)),
            pl.BlockSpec((1, _D), lambda i: (0, 0)),
        ],
        out_specs=pl.BlockSpec((_BLK, _D), lambda i: (i, 0)),
        out_shape=jax.ShapeDtypeStruct((_NP, _D), jnp.float32),
    )(agg, w, b.reshape(1, _D))


def _upd_body(am_ref, aa_ref, wi_ref, wo_ref, wu_ref,
              bi_ref, bo_ref, bu_ref, o_ref):
    m = _lrelu(jnp.dot(am_ref[0] + am_ref[1], wi_ref[...],
                       preferred_element_type=jnp.float32) + bi_ref[...])
    a = _lrelu(jnp.dot(aa_ref[0] + aa_ref[1], wo_ref[...],
                       preferred_element_type=jnp.float32) + bo_ref[...])
    u = jnp.concatenate([m, a], axis=1)
    o_ref[...] = _lrelu(jnp.dot(u, wu_ref[...],
                                preferred_element_type=jnp.float32) + bu_ref[...])


@jax.jit
def _upd(agg_m, agg_a, w_inner, w_outer, w_update, b_inner, b_outer, b_update):
    return pl.pallas_call(
        _upd_body,
        grid=(_NP // _BLK,),
        in_specs=[
            pl.BlockSpec((_SC_CORES, _BLK, _D), lambda i: (0, i, 0)),
            pl.BlockSpec((_SC_CORES, _BLK, _D), lambda i: (0, i, 0)),
            pl.BlockSpec((_D, _D), lambda i: (0, 0)),
            pl.BlockSpec((_D, _D), lambda i: (0, 0)),
            pl.BlockSpec((2 * _D, _D), lambda i: (0, 0)),
            pl.BlockSpec((1, _D), lambda i: (0, 0)),
            pl.BlockSpec((1, _D), lambda i: (0, 0)),
            pl.BlockSpec((1, _D), lambda i: (0, 0)),
        ],
        out_specs=pl.BlockSpec((_BLK, _D), lambda i: (i, 0)),
        out_shape=jax.ShapeDtypeStruct((_NP, _D), jnp.float32),
    )(agg_m, agg_a, w_inner, w_outer, w_update,
      b_inner.reshape(1, _D), b_outer.reshape(1, _D), b_update.reshape(1, _D))


def _pool_body(xi_ref, xj_ref, bi_ref, bj_ref, o_ref):
    iota = lax.broadcasted_iota(jnp.int32, (_NP // _D, _D, _B), 2)

    def pooled(x_ref, b_ref):
        oh = (b_ref[...][:, :, None] == iota).astype(jnp.float32)
        oh = oh.reshape(_NP, _B)
        s = lax.dot_general(oh, x_ref[...], (((0,), (0,)), ((), ())),
                            preferred_element_type=jnp.float32,
                            precision=lax.Precision.HIGHEST)
        cnt = jnp.sum(oh, axis=0)
        return s / jnp.clip(cnt, 1.0, None)[:, None]

    o_ref[...] = jnp.concatenate([pooled(xi_ref, bi_ref),
                                  pooled(xj_ref, bj_ref)], axis=1)


@jax.jit
def _pool(xi, xj, bi, bj, w_read, b_read):
    bi_p = jnp.pad(bi.astype(jnp.int32), (0, _NP - _N),
                   constant_values=_B).reshape(_NP // _D, _D)
    bj_p = jnp.pad(bj.astype(jnp.int32), (0, _NP - _N),
                   constant_values=_B).reshape(_NP // _D, _D)
    p = pl.pallas_call(
        _pool_body,
        out_shape=jax.ShapeDtypeStruct((_B, 2 * _D), jnp.float32),
    )(xi, xj, bi_p, bj_p)
    logits = p.reshape(_B, 1, 2 * _D) @ w_read + b_read
    return jax.nn.sigmoid(jnp.mean(logits, axis=1))


def kernel(x_i, x_j, inner_edge_index_i, inner_edge_index_j, outer_edge_index_i,
           outer_edge_index_j, x_i_batch, x_j_batch, W_enc, b_enc, W_inner, b_inner,
           W_outer, b_outer, W_update, b_update, W_read, b_read):
    x_i_p = jnp.pad(x_i.astype(jnp.float32), ((0, _NP - _N), (0, 0)))
    x_j_p = jnp.pad(x_j.astype(jnp.float32), ((0, _NP - _N), (0, 0)))
    e_ii = _edges(inner_edge_index_i)
    e_ij = _edges(inner_edge_index_j)
    e_oi = _edges(outer_edge_index_i)
    e_oj = _edges(outer_edge_index_j)
    xi = _enc(_mp(x_i_p, e_ii), W_enc, b_enc)
    xj = _enc(_mp(x_j_p, e_ij), W_enc, b_enc)
    for _ in range(2):
        am_i = _mp(xi, e_ii)
        am_j = _mp(xj, e_ij)
        aa_ij = _mp(xj, e_oj)
        aa_ji = _mp(xi, e_oi)
        xi = _upd(am_i, aa_ij, W_inner, W_outer, W_update, b_inner, b_outer, b_update)
        xj = _upd(am_j, aa_ji, W_inner, W_outer, W_update, b_inner, b_outer, b_update)
    return _pool(xi, xj, x_i_batch, x_j_batch, W_read, b_read)
